# baseline (speedup 1.0000x reference)
import functools

import jax
import jax.numpy as jnp
from jax import lax
from jax.experimental import pallas as pl
from jax.experimental.pallas import tpu as pltpu
from jax.experimental.pallas import tpu_sc as plsc

G = 64
C = 10
BLK = 1000
NEG = -1e30

S = 4
DG = 3
DS = S - DG
P = DG + 2
IS = 2 * S
CHUNK = 80


def _prop_body(nc, ns, n_nodes, iters, h_hbm, src_hbm, dst_hbm, z_hbm,
               out_hbm, isrc, idst, rows, isems, gsems, ssems, zsem, agg_sh):
    c = lax.axis_index("c")
    s = lax.axis_index("s")
    w = c * ns + s
    rmain = (n_nodes // 8 // ns) * 8
    rtail = n_nodes - ns * rmain
    zdesc = pltpu.make_async_copy(z_hbm.at[pl.ds(0, rmain)],
                                  agg_sh.at[pl.ds(s * rmain, rmain)], zsem)
    zdesc.start()
    if rtail:
        @pl.when(s == 0)
        def _():
            pltpu.sync_copy(z_hbm.at[pl.ds(0, rtail)],
                            agg_sh.at[pl.ds(ns * rmain, rtail)])

    def idescs(m, k):
        return (pltpu.make_async_copy(src_hbm.at[w, k], isrc[m], isems[m]),
                pltpu.make_async_copy(dst_hbm.at[w, k], idst[m], isems[m]))

    def istart(m, k):
        d1, d2 = idescs(m, k)
        d1.start()
        d2.start()

    def iwait(m):
        d1, d2 = idescs(m, 0)
        d1.wait()
        d2.wait()

    def gdesc(m):
        return pltpu.make_async_copy(h_hbm.at[isrc[m % IS].at[0]],
                                     rows[m % S], gsems[m % S])

    def sdesc(m):
        return pltpu.make_async_copy(rows[m % S],
                                     agg_sh.at[idst[m % IS].at[0]],
                                     ssems[m % S])

    def step(jj, m, do_swait=True, do_istart=True, do_prefetch=True):
        gdesc(m).wait()
        sdesc(m).start(add=True)
        if do_swait:
            sdesc(m - DS).wait()
        if do_istart:
            istart((m + P) % IS, jj + P)
        if do_prefetch:
            iwait((m + DG) % IS)
            gdesc(m + DG).start()

    for k in range(P):
        istart(k % IS, k)
    for t in range(DG):
        iwait(t % IS)
        gdesc(t).start()
    zdesc.wait()
    plsc.subcore_barrier()
    for t in range(DS):
        step(t, t, do_swait=False)
    lo, hi = DS, iters - P
    groups = (hi - lo) // IS

    def body(g, carry):
        base = lo + g * IS
        for b in range(IS):
            step(base + b, lo + b)
        return carry

    lax.fori_loop(0, groups, body, 0)
    for jj in range(lo + groups * IS, hi):
        step(jj, jj)
    for jj in range(hi, iters - DG):
        step(jj, jj, do_istart=False)
    for jj in range(iters - DG, iters):
        step(jj, jj, do_istart=False, do_prefetch=False)
    for jj in range(iters - DS, iters):
        sdesc(jj).wait()

    plsc.subcore_barrier()
    pltpu.sync_copy(agg_sh.at[pl.ds(s * rmain, rmain)],
                    out_hbm.at[c, pl.ds(s * rmain, rmain)])
    if rtail:
        @pl.when(s == 0)
        def _():
            pltpu.sync_copy(agg_sh.at[pl.ds(ns * rmain, rtail)],
                            out_hbm.at[c, pl.ds(ns * rmain, rtail)])


def _make_prop(n_nodes, n_edges, h):
    nc, ns = 2, 16
    mesh = plsc.VectorSubcoreMesh(core_axis_name="c", subcore_axis_name="s",
                                  num_cores=nc, num_subcores=ns)
    epw = n_edges // (nc * ns)
    iters = epw // CHUNK
    return pl.kernel(
        functools.partial(_prop_body, nc, ns, n_nodes, iters),
        out_type=jax.ShapeDtypeStruct((nc, n_nodes, h), jnp.float32),
        mesh=mesh,
        scratch_types=[
            [pltpu.VMEM((1, CHUNK), jnp.int32) for _ in range(IS)],
            [pltpu.VMEM((1, CHUNK), jnp.int32) for _ in range(IS)],
            [pltpu.VMEM((CHUNK, h), jnp.float32) for _ in range(S)],
            [pltpu.SemaphoreType.DMA for _ in range(IS)],
            [pltpu.SemaphoreType.DMA for _ in range(S)],
            [pltpu.SemaphoreType.DMA for _ in range(S)],
            pltpu.SemaphoreType.DMA,
            pltpu.VMEM_SHARED((n_nodes, h), jnp.float32),
        ],
    )


def _pre_body(x_ref, w_ref, b_ref, bat_ref, h_ref, r_ref, cnt_ref):
    i = pl.program_id(0)
    h = jnp.maximum(jnp.dot(x_ref[...], w_ref[...],
                            preferred_element_type=jnp.float32) + b_ref[...], 0.0)
    h_ref[...] = h
    onehot = (bat_ref[...] == lax.broadcasted_iota(jnp.int32, (BLK, G), 1)
              ).astype(jnp.float32)
    r = lax.dot_general(onehot, h, (((0,), (0,)), ((), ())),
                        preferred_element_type=jnp.float32)
    cnt = lax.dot_general(onehot, jnp.ones_like(h), (((0,), (0,)), ((), ())),
                          preferred_element_type=jnp.float32)

    @pl.when(i == 0)
    def _():
        r_ref[...] = jnp.zeros_like(r_ref)
        cnt_ref[...] = jnp.zeros_like(cnt_ref)

    r_ref[...] += r
    cnt_ref[...] += cnt


def _cell_body(agg_ref, w_ref, b_ref, bat_ref, h_ref, r_ref):
    i = pl.program_id(0)
    a = agg_ref[0] + agg_ref[1]
    h = jnp.maximum(jnp.dot(a, w_ref[...],
                            preferred_element_type=jnp.float32) + b_ref[...], 0.0)
    h_ref[...] = h
    onehot = (bat_ref[...] == lax.broadcasted_iota(jnp.int32, (BLK, G), 1)
              ).astype(jnp.float32)
    r = lax.dot_general(onehot, h, (((0,), (0,)), ((), ())),
                        preferred_element_type=jnp.float32)

    @pl.when(i == 0)
    def _():
        r_ref[...] = jnp.zeros_like(r_ref)

    r_ref[...] += r


def _cell3_body(agg_ref, w_ref, b_ref, bat_ref, r0_ref, r1_ref, r2_ref,
                cnt_ref, w1_ref, b1_ref, w2_ref, b2_ref, out_ref, racc):
    i = pl.program_id(0)
    ng = pl.num_programs(0)
    a = agg_ref[0] + agg_ref[1]
    h = jnp.maximum(jnp.dot(a, w_ref[...],
                            preferred_element_type=jnp.float32) + b_ref[...], 0.0)
    onehot = (bat_ref[...] == lax.broadcasted_iota(jnp.int32, (BLK, G), 1)
              ).astype(jnp.float32)
    r = lax.dot_general(onehot, h, (((0,), (0,)), ((), ())),
                        preferred_element_type=jnp.float32)

    @pl.when(i == 0)
    def _():
        racc[...] = jnp.zeros_like(racc)

    racc[...] += r

    @pl.when(i == ng - 1)
    def _():
        emb = (r0_ref[...] + r1_ref[...] + r2_ref[...] + racc[...]) \
            / jnp.maximum(cnt_ref[...], 1.0)
        t = jnp.maximum(jnp.dot(emb, w1_ref[...],
                                preferred_element_type=jnp.float32)
                        + b1_ref[...], 0.0)
        pred = jnp.dot(t, w2_ref[...],
                       preferred_element_type=jnp.float32) + b2_ref[...]
        col = lax.broadcasted_iota(jnp.int32, pred.shape, 1)
        m = jnp.where(col < C, pred, NEG)
        mx = jnp.max(m, axis=1, keepdims=True)
        lse = jnp.log(jnp.sum(jnp.exp(m - mx), axis=1, keepdims=True)) + mx
        out_ref[...] = m - lse


def _tc_pre(x, w, b, bat2, n, h):
    grid = n // BLK
    return pl.pallas_call(
        _pre_body,
        grid=(grid,),
        in_specs=[
            pl.BlockSpec((BLK, x.shape[1]), lambda i: (i, 0)),
            pl.BlockSpec((x.shape[1], h), lambda i: (0, 0)),
            pl.BlockSpec((1, h), lambda i: (0, 0)),
            pl.BlockSpec((BLK, 1), lambda i: (i, 0)),
        ],
        out_specs=[
            pl.BlockSpec((BLK, h), lambda i: (i, 0)),
            pl.BlockSpec((G, h), lambda i: (0, 0)),
            pl.BlockSpec((G, h), lambda i: (0, 0)),
        ],
        out_shape=[
            jax.ShapeDtypeStruct((n, h), jnp.float32),
            jax.ShapeDtypeStruct((G, h), jnp.float32),
            jax.ShapeDtypeStruct((G, h), jnp.float32),
        ],
    )(x, w, b, bat2)


def _tc_cell(agg, w, b, bat2, n, h):
    grid = n // BLK
    return pl.pallas_call(
        _cell_body,
        grid=(grid,),
        in_specs=[
            pl.BlockSpec((2, BLK, h), lambda i: (0, i, 0)),
            pl.BlockSpec((h, h), lambda i: (0, 0)),
            pl.BlockSpec((1, h), lambda i: (0, 0)),
            pl.BlockSpec((BLK, 1), lambda i: (i, 0)),
        ],
        out_specs=[
            pl.BlockSpec((BLK, h), lambda i: (i, 0)),
            pl.BlockSpec((G, h), lambda i: (0, 0)),
        ],
        out_shape=[
            jax.ShapeDtypeStruct((n, h), jnp.float32),
            jax.ShapeDtypeStruct((G, h), jnp.float32),
        ],
    )(agg, w, b, bat2)


def _tc_cell3(agg, w, b, bat2, r0, r1, r2, cnt, w1, b1, w2p, b2p, n, h):
    grid = n // BLK
    cm = lambda i: (0, 0)
    return pl.pallas_call(
        _cell3_body,
        grid=(grid,),
        in_specs=[
            pl.BlockSpec((2, BLK, h), lambda i: (0, i, 0)),
            pl.BlockSpec((h, h), cm),
            pl.BlockSpec((1, h), cm),
            pl.BlockSpec((BLK, 1), lambda i: (i, 0)),
            pl.BlockSpec((G, h), cm),
            pl.BlockSpec((G, h), cm),
            pl.BlockSpec((G, h), cm),
            pl.BlockSpec((G, h), cm),
            pl.BlockSpec((h, h), cm),
            pl.BlockSpec((1, h), cm),
            pl.BlockSpec((h, h), cm),
            pl.BlockSpec((1, h), cm),
        ],
        out_specs=pl.BlockSpec((G, h), cm),
        out_shape=jax.ShapeDtypeStruct((G, h), jnp.float32),
        scratch_shapes=[pltpu.VMEM((G, h), jnp.float32)],
    )(agg, w, b, bat2, r0, r1, r2, cnt, w1, b1, w2p, b2p)


def kernel(x, edge_index, batch, W_pre, b_pre, W_c0, b_c0, W_c1, b_c1,
           W_c2, b_c2, W_p1, b_p1, W_p2, b_p2):
    n, d = x.shape
    h = W_pre.shape[1]
    e = edge_index.shape[1]
    nw = 32
    epw = e // nw
    iters = epw // CHUNK
    srcr = edge_index[0].reshape(nw, iters, 1, CHUNK)
    dstr = edge_index[1].reshape(nw, iters, 1, CHUNK)
    bat2 = batch.reshape(n, 1)
    rmain = (n // 8 // 16) * 8
    zrows = jnp.zeros((rmain, h), jnp.float32)

    prop = _make_prop(n, e, h)

    hh, r0, cnt = _tc_pre(x, W_pre, b_pre.reshape(1, h), bat2, n, h)
    agg = prop(hh, srcr, dstr, zrows)
    hh, r1 = _tc_cell(agg, W_c0, b_c0.reshape(1, h), bat2, n, h)
    agg = prop(hh, srcr, dstr, zrows)
    hh, r2 = _tc_cell(agg, W_c1, b_c1.reshape(1, h), bat2, n, h)
    agg = prop(hh, srcr, dstr, zrows)

    w2p = jnp.pad(W_p2, ((0, 0), (0, h - C)))
    b2p = jnp.pad(b_p2, (0, h - C)).reshape(1, h)
    out = _tc_cell3(agg, W_c2, b_c2.reshape(1, h), bat2, r0, r1, r2, cnt,
                    W_p1, b_p1.reshape(1, h), w2p, b2p, n, h)
    return out[:, :C]


# speedup vs baseline: 1.0715x; 1.0715x over previous
import functools

import jax
import jax.numpy as jnp
from jax import lax
from jax.experimental import pallas as pl
from jax.experimental.pallas import tpu as pltpu
from jax.experimental.pallas import tpu_sc as plsc

G = 64
C = 10
BLK = 2000
NEG = -1e30

S = 4
DG = 3
DS = S - DG
P = DG + 2
IS = 2 * S
CHUNK = 80


def _prop_body(nc, ns, n_nodes, iters, h_hbm, sd_hbm, z_hbm,
               out_hbm, ibufs, rows, isems, gsems, ssems, zsem, agg_sh):
    c = lax.axis_index("c")
    s = lax.axis_index("s")
    w = c * ns + s
    rmain = (n_nodes // 8 // ns) * 8
    rtail = n_nodes - ns * rmain
    zdesc = pltpu.make_async_copy(z_hbm.at[pl.ds(0, rmain)],
                                  agg_sh.at[pl.ds(s * rmain, rmain)], zsem)
    zdesc.start()
    if rtail:
        @pl.when(s == 0)
        def _():
            pltpu.sync_copy(z_hbm.at[pl.ds(0, rtail)],
                            agg_sh.at[pl.ds(ns * rmain, rtail)])

    def idesc(m, k):
        return pltpu.make_async_copy(sd_hbm.at[w, k], ibufs[m], isems[m])

    def istart(m, k):
        idesc(m, k).start()

    def iwait(m):
        idesc(m, 0).wait()

    def gdesc(m):
        return pltpu.make_async_copy(h_hbm.at[ibufs[m % IS].at[0]],
                                     rows[m % S], gsems[m % S])

    def sdesc(m):
        return pltpu.make_async_copy(rows[m % S],
                                     agg_sh.at[ibufs[m % IS].at[1]],
                                     ssems[m % S])

    def step(jj, m, do_swait=True, do_istart=True, do_prefetch=True):
        gdesc(m).wait()
        sdesc(m).start(add=True)
        if do_swait:
            sdesc(m - DS).wait()
        if do_istart:
            istart((m + P) % IS, jj + P)
        if do_prefetch:
            iwait((m + DG) % IS)
            gdesc(m + DG).start()

    for k in range(P):
        istart(k % IS, k)
    for t in range(DG):
        iwait(t % IS)
        gdesc(t).start()
    zdesc.wait()
    plsc.subcore_barrier()
    for t in range(DS):
        step(t, t, do_swait=False)
    lo, hi = DS, iters - P
    groups = (hi - lo) // IS

    def body(g, carry):
        base = lo + g * IS
        for b in range(IS):
            step(base + b, lo + b)
        return carry

    lax.fori_loop(0, groups, body, 0)
    for jj in range(lo + groups * IS, hi):
        step(jj, jj)
    for jj in range(hi, iters - DG):
        step(jj, jj, do_istart=False)
    for jj in range(iters - DG, iters):
        step(jj, jj, do_istart=False, do_prefetch=False)
    for jj in range(iters - DS, iters):
        sdesc(jj).wait()

    plsc.subcore_barrier()
    pltpu.sync_copy(agg_sh.at[pl.ds(s * rmain, rmain)],
                    out_hbm.at[c, pl.ds(s * rmain, rmain)])
    if rtail:
        @pl.when(s == 0)
        def _():
            pltpu.sync_copy(agg_sh.at[pl.ds(ns * rmain, rtail)],
                            out_hbm.at[c, pl.ds(ns * rmain, rtail)])


def _make_prop(n_nodes, n_edges, h):
    nc, ns = 2, 16
    mesh = plsc.VectorSubcoreMesh(core_axis_name="c", subcore_axis_name="s",
                                  num_cores=nc, num_subcores=ns)
    epw = n_edges // (nc * ns)
    iters = epw // CHUNK
    return pl.kernel(
        functools.partial(_prop_body, nc, ns, n_nodes, iters),
        out_type=jax.ShapeDtypeStruct((nc, n_nodes, h), jnp.float32),
        mesh=mesh,
        scratch_types=[
            [pltpu.VMEM((2, CHUNK), jnp.int32) for _ in range(IS)],
            [pltpu.VMEM((CHUNK, h), jnp.float32) for _ in range(S)],
            [pltpu.SemaphoreType.DMA for _ in range(IS)],
            [pltpu.SemaphoreType.DMA for _ in range(S)],
            [pltpu.SemaphoreType.DMA for _ in range(S)],
            pltpu.SemaphoreType.DMA,
            pltpu.VMEM_SHARED((n_nodes, h), jnp.float32),
        ],
    )


def _pre_body(x_ref, w_ref, b_ref, bat_ref, h_ref, r_ref, cnt_ref):
    i = pl.program_id(0)
    h = jnp.maximum(jnp.dot(x_ref[...], w_ref[...],
                            preferred_element_type=jnp.float32) + b_ref[...], 0.0)
    h_ref[...] = h
    onehot = (bat_ref[...] == lax.broadcasted_iota(jnp.int32, (BLK, G), 1)
              ).astype(jnp.float32)
    r = lax.dot_general(onehot, h, (((0,), (0,)), ((), ())),
                        preferred_element_type=jnp.float32)
    cnt = lax.dot_general(onehot, jnp.ones_like(h), (((0,), (0,)), ((), ())),
                          preferred_element_type=jnp.float32)

    @pl.when(i == 0)
    def _():
        r_ref[...] = jnp.zeros_like(r_ref)
        cnt_ref[...] = jnp.zeros_like(cnt_ref)

    r_ref[...] += r
    cnt_ref[...] += cnt


def _cell_body(agg_ref, w_ref, b_ref, bat_ref, h_ref, r_ref):
    i = pl.program_id(0)
    a = agg_ref[0] + agg_ref[1]
    h = jnp.maximum(jnp.dot(a, w_ref[...],
                            preferred_element_type=jnp.float32) + b_ref[...], 0.0)
    h_ref[...] = h
    onehot = (bat_ref[...] == lax.broadcasted_iota(jnp.int32, (BLK, G), 1)
              ).astype(jnp.float32)
    r = lax.dot_general(onehot, h, (((0,), (0,)), ((), ())),
                        preferred_element_type=jnp.float32)

    @pl.when(i == 0)
    def _():
        r_ref[...] = jnp.zeros_like(r_ref)

    r_ref[...] += r


def _cell3_body(agg_ref, w_ref, b_ref, bat_ref, r0_ref, r1_ref, r2_ref,
                cnt_ref, w1_ref, b1_ref, w2_ref, b2_ref, out_ref, racc):
    i = pl.program_id(0)
    ng = pl.num_programs(0)
    a = agg_ref[0] + agg_ref[1]
    h = jnp.maximum(jnp.dot(a, w_ref[...],
                            preferred_element_type=jnp.float32) + b_ref[...], 0.0)
    onehot = (bat_ref[...] == lax.broadcasted_iota(jnp.int32, (BLK, G), 1)
              ).astype(jnp.float32)
    r = lax.dot_general(onehot, h, (((0,), (0,)), ((), ())),
                        preferred_element_type=jnp.float32)

    @pl.when(i == 0)
    def _():
        racc[...] = jnp.zeros_like(racc)

    racc[...] += r

    @pl.when(i == ng - 1)
    def _():
        emb = (r0_ref[...] + r1_ref[...] + r2_ref[...] + racc[...]) \
            / jnp.maximum(cnt_ref[...], 1.0)
        t = jnp.maximum(jnp.dot(emb, w1_ref[...],
                                preferred_element_type=jnp.float32)
                        + b1_ref[...], 0.0)
        pred = jnp.dot(t, w2_ref[...],
                       preferred_element_type=jnp.float32) + b2_ref[...]
        col = lax.broadcasted_iota(jnp.int32, pred.shape, 1)
        m = jnp.where(col < C, pred, NEG)
        mx = jnp.max(m, axis=1, keepdims=True)
        lse = jnp.log(jnp.sum(jnp.exp(m - mx), axis=1, keepdims=True)) + mx
        out_ref[...] = m - lse


def _tc_pre(x, w, b, bat2, n, h):
    grid = n // BLK
    return pl.pallas_call(
        _pre_body,
        grid=(grid,),
        in_specs=[
            pl.BlockSpec((BLK, x.shape[1]), lambda i: (i, 0)),
            pl.BlockSpec((x.shape[1], h), lambda i: (0, 0)),
            pl.BlockSpec((1, h), lambda i: (0, 0)),
            pl.BlockSpec((BLK, 1), lambda i: (i, 0)),
        ],
        out_specs=[
            pl.BlockSpec((BLK, h), lambda i: (i, 0)),
            pl.BlockSpec((G, h), lambda i: (0, 0)),
            pl.BlockSpec((G, h), lambda i: (0, 0)),
        ],
        out_shape=[
            jax.ShapeDtypeStruct((n, h), jnp.float32),
            jax.ShapeDtypeStruct((G, h), jnp.float32),
            jax.ShapeDtypeStruct((G, h), jnp.float32),
        ],
    )(x, w, b, bat2)


def _tc_cell(agg, w, b, bat2, n, h):
    grid = n // BLK
    return pl.pallas_call(
        _cell_body,
        grid=(grid,),
        in_specs=[
            pl.BlockSpec((2, BLK, h), lambda i: (0, i, 0)),
            pl.BlockSpec((h, h), lambda i: (0, 0)),
            pl.BlockSpec((1, h), lambda i: (0, 0)),
            pl.BlockSpec((BLK, 1), lambda i: (i, 0)),
        ],
        out_specs=[
            pl.BlockSpec((BLK, h), lambda i: (i, 0)),
            pl.BlockSpec((G, h), lambda i: (0, 0)),
        ],
        out_shape=[
            jax.ShapeDtypeStruct((n, h), jnp.float32),
            jax.ShapeDtypeStruct((G, h), jnp.float32),
        ],
    )(agg, w, b, bat2)


def _tc_cell3(agg, w, b, bat2, r0, r1, r2, cnt, w1, b1, w2p, b2p, n, h):
    grid = n // BLK
    cm = lambda i: (0, 0)
    return pl.pallas_call(
        _cell3_body,
        grid=(grid,),
        in_specs=[
            pl.BlockSpec((2, BLK, h), lambda i: (0, i, 0)),
            pl.BlockSpec((h, h), cm),
            pl.BlockSpec((1, h), cm),
            pl.BlockSpec((BLK, 1), lambda i: (i, 0)),
            pl.BlockSpec((G, h), cm),
            pl.BlockSpec((G, h), cm),
            pl.BlockSpec((G, h), cm),
            pl.BlockSpec((G, h), cm),
            pl.BlockSpec((h, h), cm),
            pl.BlockSpec((1, h), cm),
            pl.BlockSpec((h, h), cm),
            pl.BlockSpec((1, h), cm),
        ],
        out_specs=pl.BlockSpec((G, h), cm),
        out_shape=jax.ShapeDtypeStruct((G, h), jnp.float32),
        scratch_shapes=[pltpu.VMEM((G, h), jnp.float32)],
    )(agg, w, b, bat2, r0, r1, r2, cnt, w1, b1, w2p, b2p)


def kernel(x, edge_index, batch, W_pre, b_pre, W_c0, b_c0, W_c1, b_c1,
           W_c2, b_c2, W_p1, b_p1, W_p2, b_p2):
    n, d = x.shape
    h = W_pre.shape[1]
    e = edge_index.shape[1]
    nw = 32
    epw = e // nw
    iters = epw // CHUNK
    sd = edge_index.reshape(2, nw, iters, CHUNK).transpose(1, 2, 0, 3)
    bat2 = batch.reshape(n, 1)
    rmain = (n // 8 // 16) * 8
    zrows = jnp.zeros((rmain, h), jnp.float32)

    prop = _make_prop(n, e, h)

    hh, r0, cnt = _tc_pre(x, W_pre, b_pre.reshape(1, h), bat2, n, h)
    agg = prop(hh, sd, zrows)
    hh, r1 = _tc_cell(agg, W_c0, b_c0.reshape(1, h), bat2, n, h)
    agg = prop(hh, sd, zrows)
    hh, r2 = _tc_cell(agg, W_c1, b_c1.reshape(1, h), bat2, n, h)
    agg = prop(hh, sd, zrows)

    w2p = jnp.pad(W_p2, ((0, 0), (0, h - C)))
    b2p = jnp.pad(b_p2, (0, h - C)).reshape(1, h)
    out = _tc_cell3(agg, W_c2, b_c2.reshape(1, h), bat2, r0, r1, r2, cnt,
                    W_p1, b_p1.reshape(1, h), w2p, b2p, n, h)
    return out[:, :C]


# speedup vs baseline: 1.0725x; 1.0010x over previous
import functools

import jax
import jax.numpy as jnp
from jax import lax
from jax.experimental import pallas as pl
from jax.experimental.pallas import tpu as pltpu
from jax.experimental.pallas import tpu_sc as plsc

G = 64
C = 10
BLK = 5000
NEG = -1e30

S = 4
DG = 3
DS = S - DG
P = DG + 3
IS = 2 * S
CHUNK = 80


def _prop_body(nc, ns, n_nodes, iters, h_hbm, sd_hbm, z_hbm,
               out_hbm, ibufs, rows, isems, gsems, ssems, zsem, agg_sh):
    c = lax.axis_index("c")
    s = lax.axis_index("s")
    w = c * ns + s
    rmain = (n_nodes // 8 // ns) * 8
    rtail = n_nodes - ns * rmain
    zdesc = pltpu.make_async_copy(z_hbm.at[pl.ds(0, rmain)],
                                  agg_sh.at[pl.ds(s * rmain, rmain)], zsem)
    zdesc.start()
    if rtail:
        @pl.when(s == 0)
        def _():
            pltpu.sync_copy(z_hbm.at[pl.ds(0, rtail)],
                            agg_sh.at[pl.ds(ns * rmain, rtail)])

    def idesc(m, k):
        return pltpu.make_async_copy(sd_hbm.at[w, k], ibufs[m], isems[m])

    def istart(m, k):
        idesc(m, k).start()

    def iwait(m):
        idesc(m, 0).wait()

    def gdesc(m):
        return pltpu.make_async_copy(h_hbm.at[ibufs[m % IS].at[0]],
                                     rows[m % S], gsems[m % S])

    def sdesc(m):
        return pltpu.make_async_copy(rows[m % S],
                                     agg_sh.at[ibufs[m % IS].at[1]],
                                     ssems[m % S])

    def step(jj, m, do_swait=True, do_istart=True, do_prefetch=True):
        gdesc(m).wait()
        sdesc(m).start(add=True)
        if do_swait:
            sdesc(m - DS).wait()
        if do_istart:
            istart((m + P) % IS, jj + P)
        if do_prefetch:
            iwait((m + DG) % IS)
            gdesc(m + DG).start()

    for k in range(P):
        istart(k % IS, k)
    for t in range(DG):
        iwait(t % IS)
        gdesc(t).start()
    zdesc.wait()
    plsc.subcore_barrier()
    for t in range(DS):
        step(t, t, do_swait=False)
    lo, hi = DS, iters - P
    groups = (hi - lo) // IS

    def body(g, carry):
        base = lo + g * IS
        for b in range(IS):
            step(base + b, lo + b)
        return carry

    lax.fori_loop(0, groups, body, 0)
    for jj in range(lo + groups * IS, hi):
        step(jj, jj)
    for jj in range(hi, iters - DG):
        step(jj, jj, do_istart=False)
    for jj in range(iters - DG, iters):
        step(jj, jj, do_istart=False, do_prefetch=False)
    for jj in range(iters - DS, iters):
        sdesc(jj).wait()

    plsc.subcore_barrier()
    pltpu.sync_copy(agg_sh.at[pl.ds(s * rmain, rmain)],
                    out_hbm.at[c, pl.ds(s * rmain, rmain)])
    if rtail:
        @pl.when(s == 0)
        def _():
            pltpu.sync_copy(agg_sh.at[pl.ds(ns * rmain, rtail)],
                            out_hbm.at[c, pl.ds(ns * rmain, rtail)])


def _make_prop(n_nodes, n_edges, h):
    nc, ns = 2, 16
    mesh = plsc.VectorSubcoreMesh(core_axis_name="c", subcore_axis_name="s",
                                  num_cores=nc, num_subcores=ns)
    epw = n_edges // (nc * ns)
    iters = epw // CHUNK
    return pl.kernel(
        functools.partial(_prop_body, nc, ns, n_nodes, iters),
        out_type=jax.ShapeDtypeStruct((nc, n_nodes, h), jnp.float32),
        mesh=mesh,
        scratch_types=[
            [pltpu.VMEM((2, CHUNK), jnp.int32) for _ in range(IS)],
            [pltpu.VMEM((CHUNK, h), jnp.float32) for _ in range(S)],
            [pltpu.SemaphoreType.DMA for _ in range(IS)],
            [pltpu.SemaphoreType.DMA for _ in range(S)],
            [pltpu.SemaphoreType.DMA for _ in range(S)],
            pltpu.SemaphoreType.DMA,
            pltpu.VMEM_SHARED((n_nodes, h), jnp.float32),
        ],
    )


def _pre_body(x_ref, w_ref, b_ref, bat_ref, h_ref, r_ref, cnt_ref):
    i = pl.program_id(0)
    h = jnp.maximum(jnp.dot(x_ref[...], w_ref[...],
                            preferred_element_type=jnp.float32) + b_ref[...], 0.0)
    h_ref[...] = h
    onehot = (bat_ref[...] == lax.broadcasted_iota(jnp.int32, (BLK, G), 1)
              ).astype(jnp.float32)
    r = lax.dot_general(onehot, h, (((0,), (0,)), ((), ())),
                        preferred_element_type=jnp.float32)
    cnt = lax.dot_general(onehot, jnp.ones_like(h), (((0,), (0,)), ((), ())),
                          preferred_element_type=jnp.float32)

    @pl.when(i == 0)
    def _():
        r_ref[...] = jnp.zeros_like(r_ref)
        cnt_ref[...] = jnp.zeros_like(cnt_ref)

    r_ref[...] += r
    cnt_ref[...] += cnt


def _cell_body(agg_ref, w_ref, b_ref, bat_ref, h_ref, r_ref):
    i = pl.program_id(0)
    a = agg_ref[0] + agg_ref[1]
    h = jnp.maximum(jnp.dot(a, w_ref[...],
                            preferred_element_type=jnp.float32) + b_ref[...], 0.0)
    h_ref[...] = h
    onehot = (bat_ref[...] == lax.broadcasted_iota(jnp.int32, (BLK, G), 1)
              ).astype(jnp.float32)
    r = lax.dot_general(onehot, h, (((0,), (0,)), ((), ())),
                        preferred_element_type=jnp.float32)

    @pl.when(i == 0)
    def _():
        r_ref[...] = jnp.zeros_like(r_ref)

    r_ref[...] += r


def _cell3_body(agg_ref, w_ref, b_ref, bat_ref, r0_ref, r1_ref, r2_ref,
                cnt_ref, w1_ref, b1_ref, w2_ref, b2_ref, out_ref, racc):
    i = pl.program_id(0)
    ng = pl.num_programs(0)
    a = agg_ref[0] + agg_ref[1]
    h = jnp.maximum(jnp.dot(a, w_ref[...],
                            preferred_element_type=jnp.float32) + b_ref[...], 0.0)
    onehot = (bat_ref[...] == lax.broadcasted_iota(jnp.int32, (BLK, G), 1)
              ).astype(jnp.float32)
    r = lax.dot_general(onehot, h, (((0,), (0,)), ((), ())),
                        preferred_element_type=jnp.float32)

    @pl.when(i == 0)
    def _():
        racc[...] = jnp.zeros_like(racc)

    racc[...] += r

    @pl.when(i == ng - 1)
    def _():
        emb = (r0_ref[...] + r1_ref[...] + r2_ref[...] + racc[...]) \
            / jnp.maximum(cnt_ref[...], 1.0)
        t = jnp.maximum(jnp.dot(emb, w1_ref[...],
                                preferred_element_type=jnp.float32)
                        + b1_ref[...], 0.0)
        pred = jnp.dot(t, w2_ref[...],
                       preferred_element_type=jnp.float32) + b2_ref[...]
        col = lax.broadcasted_iota(jnp.int32, pred.shape, 1)
        m = jnp.where(col < C, pred, NEG)
        mx = jnp.max(m, axis=1, keepdims=True)
        lse = jnp.log(jnp.sum(jnp.exp(m - mx), axis=1, keepdims=True)) + mx
        out_ref[...] = m - lse


def _tc_pre(x, w, b, bat2, n, h):
    grid = n // BLK
    return pl.pallas_call(
        _pre_body,
        grid=(grid,),
        in_specs=[
            pl.BlockSpec((BLK, x.shape[1]), lambda i: (i, 0)),
            pl.BlockSpec((x.shape[1], h), lambda i: (0, 0)),
            pl.BlockSpec((1, h), lambda i: (0, 0)),
            pl.BlockSpec((BLK, 1), lambda i: (i, 0)),
        ],
        out_specs=[
            pl.BlockSpec((BLK, h), lambda i: (i, 0)),
            pl.BlockSpec((G, h), lambda i: (0, 0)),
            pl.BlockSpec((G, h), lambda i: (0, 0)),
        ],
        out_shape=[
            jax.ShapeDtypeStruct((n, h), jnp.float32),
            jax.ShapeDtypeStruct((G, h), jnp.float32),
            jax.ShapeDtypeStruct((G, h), jnp.float32),
        ],
    )(x, w, b, bat2)


def _tc_cell(agg, w, b, bat2, n, h):
    grid = n // BLK
    return pl.pallas_call(
        _cell_body,
        grid=(grid,),
        in_specs=[
            pl.BlockSpec((2, BLK, h), lambda i: (0, i, 0)),
            pl.BlockSpec((h, h), lambda i: (0, 0)),
            pl.BlockSpec((1, h), lambda i: (0, 0)),
            pl.BlockSpec((BLK, 1), lambda i: (i, 0)),
        ],
        out_specs=[
            pl.BlockSpec((BLK, h), lambda i: (i, 0)),
            pl.BlockSpec((G, h), lambda i: (0, 0)),
        ],
        out_shape=[
            jax.ShapeDtypeStruct((n, h), jnp.float32),
            jax.ShapeDtypeStruct((G, h), jnp.float32),
        ],
    )(agg, w, b, bat2)


def _tc_cell3(agg, w, b, bat2, r0, r1, r2, cnt, w1, b1, w2p, b2p, n, h):
    grid = n // BLK
    cm = lambda i: (0, 0)
    return pl.pallas_call(
        _cell3_body,
        grid=(grid,),
        in_specs=[
            pl.BlockSpec((2, BLK, h), lambda i: (0, i, 0)),
            pl.BlockSpec((h, h), cm),
            pl.BlockSpec((1, h), cm),
            pl.BlockSpec((BLK, 1), lambda i: (i, 0)),
            pl.BlockSpec((G, h), cm),
            pl.BlockSpec((G, h), cm),
            pl.BlockSpec((G, h), cm),
            pl.BlockSpec((G, h), cm),
            pl.BlockSpec((h, h), cm),
            pl.BlockSpec((1, h), cm),
            pl.BlockSpec((h, h), cm),
            pl.BlockSpec((1, h), cm),
        ],
        out_specs=pl.BlockSpec((G, h), cm),
        out_shape=jax.ShapeDtypeStruct((G, h), jnp.float32),
        scratch_shapes=[pltpu.VMEM((G, h), jnp.float32)],
    )(agg, w, b, bat2, r0, r1, r2, cnt, w1, b1, w2p, b2p)


def kernel(x, edge_index, batch, W_pre, b_pre, W_c0, b_c0, W_c1, b_c1,
           W_c2, b_c2, W_p1, b_p1, W_p2, b_p2):
    n, d = x.shape
    h = W_pre.shape[1]
    e = edge_index.shape[1]
    nw = 32
    epw = e // nw
    iters = epw // CHUNK
    sd = edge_index.reshape(2, nw, iters, CHUNK).transpose(1, 2, 0, 3)
    bat2 = batch.reshape(n, 1)
    rmain = (n // 8 // 16) * 8
    zrows = jnp.zeros((rmain, h), jnp.float32)

    prop = _make_prop(n, e, h)

    hh, r0, cnt = _tc_pre(x, W_pre, b_pre.reshape(1, h), bat2, n, h)
    agg = prop(hh, sd, zrows)
    hh, r1 = _tc_cell(agg, W_c0, b_c0.reshape(1, h), bat2, n, h)
    agg = prop(hh, sd, zrows)
    hh, r2 = _tc_cell(agg, W_c1, b_c1.reshape(1, h), bat2, n, h)
    agg = prop(hh, sd, zrows)

    w2p = jnp.pad(W_p2, ((0, 0), (0, h - C)))
    b2p = jnp.pad(b_p2, (0, h - C)).reshape(1, h)
    out = _tc_cell3(agg, W_c2, b_c2.reshape(1, h), bat2, r0, r1, r2, cnt,
                    W_p1, b_p1.reshape(1, h), w2p, b2p, n, h)
    return out[:, :C]
